# baseline (speedup 1.0000x reference)
import functools

import jax
import jax.numpy as jnp
from jax import lax
from jax.experimental import pallas as pl
from jax.experimental.pallas import tpu as pltpu
from jax.experimental.pallas import tpu_sc as plsc

B, N, C, K, BINS, M, O = 4, 10000, 32, 32, 8, 1, 32
PTS = B * N
NC, NS = 2, 16
NW = NC * NS
PW = 1280
BLK = 4
ROWS = BLK * K
BLOCKS = PW // BLK
RBLOCKS = PTS // BLK
NBUF = 4
CHUNK_BLKS = 32
CHUNK_PTS = CHUNK_BLKS * BLK
NCHUNK = BLOCKS // CHUNK_BLKS
HBM_CORE = 0


def _sc_dw_kernel(table, idx, filt, cnt, dk, rcp, out,
                  dkf_v, idx_v0, idx_v1, filt_v0, filt_v1, cnt_v0, cnt_v1,
                  rcp_v, rows_v, out_v, tab_sh, sem_c0, sem_c1, *sems):
    c = lax.axis_index("c")
    s = lax.axis_index("s")
    wid = s * NC + c
    blk_base = wid * BLOCKS

    idx_b = (idx_v0, idx_v1)
    filt_b = (filt_v0, filt_v1)
    cnt_b = (cnt_v0, cnt_v1)
    csem = (sem_c0, sem_c1)

    def chunk_base(ci):
        return jnp.minimum(blk_base + ci * CHUNK_BLKS, RBLOCKS - CHUNK_BLKS)

    def chunk_load(ci, par):
        gb = chunk_base(ci)
        pltpu.async_copy(idx.at[pl.ds(gb, CHUNK_BLKS)], idx_b[par], csem[par])
        pltpu.async_copy(filt.at[pl.ds(gb, CHUNK_BLKS)], filt_b[par], csem[par])
        pltpu.async_copy(cnt.at[pl.ds(gb * BLK, CHUNK_PTS)],
                         cnt_b[par].at[pl.ds(0, CHUNK_PTS)], csem[par])

    def chunk_wait(par):
        pltpu.make_async_copy(idx.at[pl.ds(0, CHUNK_BLKS)], idx_b[par],
                              csem[par]).wait()
        pltpu.make_async_copy(filt.at[pl.ds(0, CHUNK_BLKS)], filt_b[par],
                              csem[par]).wait()
        pltpu.make_async_copy(cnt.at[pl.ds(0, CHUNK_PTS)],
                              cnt_b[par].at[pl.ds(0, CHUNK_PTS)],
                              csem[par]).wait()

    def gather(idxb, b, j, sem):
        @pl.when(c == HBM_CORE)
        def _():
            pltpu.async_copy(table.at[idxb.at[b]], rows_v.at[j], sem)

        @pl.when(c != HBM_CORE)
        def _():
            pltpu.async_copy(tab_sh.at[idxb.at[b]], rows_v.at[j], sem)

    def gather_wait(idxb, b, j, sem):
        pltpu.make_async_copy(table.at[idxb.at[b]], rows_v.at[j], sem).wait()

    iota0 = lax.iota(jnp.int32, 16)
    iota1 = iota0 + 16
    gdn = lax.GatherDimensionNumbers(offset_dims=(), collapsed_slice_dims=(0,),
                                     start_index_map=(0,))

    def vsplat(vec, lane):
        idx = jnp.full((16, 1), lane, dtype=jnp.int32)
        return lax.gather(vec, idx, gdn, slice_sizes=(1,),
                          mode=lax.GatherScatterMode.PROMISE_IN_BOUNDS)

    def compute_block(b, j, filtb, cntb):
        cl = cntb[pl.ds(b * BLK, 16)]
        cidx = jnp.clip(jnp.maximum(cl, 1) - 1, 0, K - 1)
        invv = plsc.load_gather(rcp_v, [cidx])
        for p in range(BLK):
            fv0 = filtb[b, pl.ds(p * K, 16)]
            fv1 = filtb[b, pl.ds(p * K + 16, 16)]
            acc0 = jnp.zeros((16,), jnp.float32)
            acc1 = jnp.zeros((16,), jnp.float32)
            acc2 = jnp.zeros((16,), jnp.float32)
            acc3 = jnp.zeros((16,), jnp.float32)
            for k in range(K):
                r = p * K + k
                b32 = vsplat(fv0 if k < 16 else fv1, k % 16) * C
                w0 = plsc.load_gather(dkf_v, [b32 + iota0])
                w1 = plsc.load_gather(dkf_v, [b32 + iota1])
                if k % 2 == 0:
                    acc0 = acc0 + rows_v[j, r, pl.ds(0, 16)] * w0
                    acc1 = acc1 + rows_v[j, r, pl.ds(16, 16)] * w1
                else:
                    acc2 = acc2 + rows_v[j, r, pl.ds(0, 16)] * w0
                    acc3 = acc3 + rows_v[j, r, pl.ds(16, 16)] * w1
            acc0 = acc0 + acc2
            acc1 = acc1 + acc3
            inv = vsplat(invv, p)
            out_v[b, pl.ds(p * C, 16)] = acc0 * inv
            out_v[b, pl.ds(p * C + 16, 16)] = acc1 * inv

    def do_chunk(ci, parc):
        chunk_wait(parc)

        @pl.when(ci + 1 < NCHUNK)
        def _():
            chunk_load(ci + 1, 1 - parc)

        idxb, filtb, cntb = idx_b[parc], filt_b[parc], cnt_b[parc]
        for j in range(NBUF - 1):
            gather(idxb, j, j, sems[j])

        def body(bb, _):
            for j in range(NBUF):
                b = NBUF * bb + j
                nj = (j + NBUF - 1) % NBUF

                @pl.when(b + NBUF - 1 < CHUNK_BLKS)
                def _():
                    gather(idxb, b + NBUF - 1, nj, sems[nj])

                gather_wait(idxb, b, j, sems[j])
                compute_block(b, j, filtb, cntb)
            return 0

        lax.fori_loop(0, CHUNK_BLKS // NBUF, body, 0)
        pltpu.sync_copy(out_v, out.at[pl.ds(chunk_base(ci), CHUNK_BLKS)])

    chunk_load(0, 0)
    shard = PTS // NS

    @pl.when(c != HBM_CORE)
    def _():
        pltpu.sync_copy(table.at[pl.ds(s * shard, shard)],
                        tab_sh.at[pl.ds(s * shard, shard)])

    pltpu.sync_copy(dk.at[:], dkf_v)
    pltpu.sync_copy(rcp.at[:], rcp_v)
    plsc.subcore_barrier()

    def cpair(cc, _):
        do_chunk(2 * cc, 0)
        do_chunk(2 * cc + 1, 1)
        return 0

    lax.fori_loop(0, NCHUNK // 2, cpair, 0)


def _make_sc_call():
    mesh = plsc.VectorSubcoreMesh(core_axis_name="c", subcore_axis_name="s",
                                  num_cores=NC, num_subcores=NS)
    return pl.kernel(
        _sc_dw_kernel,
        out_type=jax.ShapeDtypeStruct((RBLOCKS, BLK * C), jnp.float32),
        mesh=mesh,
        compiler_params=pltpu.CompilerParams(use_tc_tiling_on_sc=False,
                                             needs_layout_passes=False),
        scratch_types=[
            pltpu.VMEM((BINS * C,), jnp.float32),
            pltpu.VMEM((CHUNK_BLKS, ROWS), jnp.int32),
            pltpu.VMEM((CHUNK_BLKS, ROWS), jnp.int32),
            pltpu.VMEM((CHUNK_BLKS, ROWS), jnp.int32),
            pltpu.VMEM((CHUNK_BLKS, ROWS), jnp.int32),
            pltpu.VMEM((CHUNK_PTS + 16,), jnp.int32),
            pltpu.VMEM((CHUNK_PTS + 16,), jnp.int32),
            pltpu.VMEM((K,), jnp.float32),
            pltpu.VMEM((NBUF, ROWS, C), jnp.float32),
            pltpu.VMEM((CHUNK_BLKS, BLK * C), jnp.float32),
            pltpu.VMEM_SHARED((PTS, C), jnp.float32),
            pltpu.SemaphoreType.DMA,
            pltpu.SemaphoreType.DMA,
        ] + [pltpu.SemaphoreType.DMA] * NBUF,
    )


PACK = BLK
PROWS = PTS // PACK


def _fold4(x):
    s = x[:, 0:O] + x[:, O:2 * O] + x[:, 2 * O:3 * O] + x[:, 3 * O:4 * O]
    return s, jnp.concatenate([s, s, s, s], axis=1)


def _tc_body(dw_ref, w_ref, b_ref, g_ref, be_ref, y_ref):
    x = dw_ref[:, :]
    y = jnp.dot(x, w_ref[:, :], preferred_element_type=jnp.float32) + b_ref[:, :]
    _, m = _fold4(jnp.sum(y, axis=0, keepdims=True) * (1.0 / PTS))
    d = y - m
    _, v = _fold4(jnp.sum(d * d, axis=0, keepdims=True) * (1.0 / PTS))
    scale = g_ref[:, :] / jnp.sqrt(v + 1e-5)
    y_ref[:, :] = jnp.maximum(d * scale + be_ref[:, :], 0.0)


def kernel(inputs, nn_index, nn_count, filt_index, depthwise_kernel, fc_w, fc_b, gamma, beta):
    table = inputs.reshape(PTS, C)
    offs = (jnp.arange(B, dtype=jnp.int32) * N)[:, None, None]
    idx2 = (nn_index.reshape(B, N * K // ROWS, ROWS) + offs).reshape(RBLOCKS, ROWS)
    filt2 = filt_index.reshape(RBLOCKS, ROWS)
    cnt1 = nn_count.reshape(PTS)
    dk2 = depthwise_kernel.reshape(BINS * C * M)
    rcp = 1.0 / jnp.arange(1, K + 1, dtype=jnp.float32)

    dw = _make_sc_call()(table, idx2, filt2, cnt1, dk2, rcp)

    w_bd = jnp.kron(jnp.eye(PACK, dtype=jnp.float32), fc_w)
    b_t = jnp.tile(fc_b, PACK).reshape(1, PACK * O)
    g_t = jnp.tile(gamma, PACK).reshape(1, PACK * O)
    be_t = jnp.tile(beta, PACK).reshape(1, PACK * O)
    y = pl.pallas_call(
        _tc_body,
        out_shape=jax.ShapeDtypeStruct((PROWS, PACK * O), jnp.float32),
    )(dw, w_bd, b_t, g_t, be_t)
    return y.reshape(B, N, O)


# speedup vs baseline: 1.2668x; 1.2668x over previous
import functools

import jax
import jax.numpy as jnp
from jax import lax
from jax.experimental import pallas as pl
from jax.experimental.pallas import tpu as pltpu
from jax.experimental.pallas import tpu_sc as plsc

B, N, C, K, BINS, M, O = 4, 10000, 32, 32, 8, 1, 32
PTS = B * N
NC, NS = 2, 16
NW = NC * NS
PW = 1280
BLK = 4
ROWS = BLK * K
BLOCKS = PW // BLK
RBLOCKS = PTS // BLK
NBUF = 4
CHUNK_BLKS = 32
CHUNK_PTS = CHUNK_BLKS * BLK
NCHUNK = BLOCKS // CHUNK_BLKS
HBM_CORE = 0


def _sc_dw_kernel(table, idx, filt, cnt, dk, rcp, out,
                  dkf_v, idx_v0, idx_v1, filt_v0, filt_v1, cnt_v0, cnt_v1,
                  rcp_v, rows_v, out_v, tab_sh, sem_c0, sem_c1, *sems):
    c = lax.axis_index("c")
    s = lax.axis_index("s")
    wid = s * NC + c
    blk_base = wid * BLOCKS

    idx_b = (idx_v0, idx_v1)
    filt_b = (filt_v0, filt_v1)
    cnt_b = (cnt_v0, cnt_v1)
    csem = (sem_c0, sem_c1)

    def chunk_base(ci):
        return jnp.minimum(blk_base + ci * CHUNK_BLKS, RBLOCKS - CHUNK_BLKS)

    def chunk_load(ci, par):
        gb = chunk_base(ci)
        pltpu.async_copy(idx.at[pl.ds(gb, CHUNK_BLKS)], idx_b[par], csem[par])
        pltpu.async_copy(filt.at[pl.ds(gb, CHUNK_BLKS)], filt_b[par], csem[par])
        pltpu.async_copy(cnt.at[pl.ds(gb * BLK, CHUNK_PTS)],
                         cnt_b[par].at[pl.ds(0, CHUNK_PTS)], csem[par])

    def chunk_wait(par):
        pltpu.make_async_copy(idx.at[pl.ds(0, CHUNK_BLKS)], idx_b[par],
                              csem[par]).wait()
        pltpu.make_async_copy(filt.at[pl.ds(0, CHUNK_BLKS)], filt_b[par],
                              csem[par]).wait()
        pltpu.make_async_copy(cnt.at[pl.ds(0, CHUNK_PTS)],
                              cnt_b[par].at[pl.ds(0, CHUNK_PTS)],
                              csem[par]).wait()

    def gather(idxb, b, j, sem):
        @pl.when(c == HBM_CORE)
        def _():
            pltpu.async_copy(table.at[idxb.at[b]], rows_v.at[j], sem)

        @pl.when(c != HBM_CORE)
        def _():
            pltpu.async_copy(tab_sh.at[idxb.at[b]], rows_v.at[j], sem)

    def gather_wait(idxb, b, j, sem):
        pltpu.make_async_copy(table.at[idxb.at[b]], rows_v.at[j], sem).wait()

    iota0 = lax.iota(jnp.int32, 16)
    iota1 = iota0 + 16
    gdn = lax.GatherDimensionNumbers(offset_dims=(), collapsed_slice_dims=(0,),
                                     start_index_map=(0,))

    def vsplat(vec, lane):
        idx = jnp.full((16, 1), lane, dtype=jnp.int32)
        return lax.gather(vec, idx, gdn, slice_sizes=(1,),
                          mode=lax.GatherScatterMode.PROMISE_IN_BOUNDS)

    def compute_block(b, j, filtb, cntb):
        cl = cntb[pl.ds(b * BLK, 16)]
        cidx = jnp.clip(jnp.maximum(cl, 1) - 1, 0, K - 1)
        invv = plsc.load_gather(rcp_v, [cidx])
        for p in range(BLK):
            fv0 = filtb[b, pl.ds(p * K, 16)]
            fv1 = filtb[b, pl.ds(p * K + 16, 16)]
            acc0 = jnp.zeros((16,), jnp.float32)
            acc1 = jnp.zeros((16,), jnp.float32)
            for k in range(K):
                r = p * K + k
                b32 = vsplat(fv0 if k < 16 else fv1, k % 16) * C
                w0 = plsc.load_gather(dkf_v, [b32 + iota0])
                w1 = plsc.load_gather(dkf_v, [b32 + iota1])
                acc0 = acc0 + rows_v[j, r, pl.ds(0, 16)] * w0
                acc1 = acc1 + rows_v[j, r, pl.ds(16, 16)] * w1
            inv = vsplat(invv, p)
            out_v[b, pl.ds(p * C, 16)] = acc0 * inv
            out_v[b, pl.ds(p * C + 16, 16)] = acc1 * inv

    def do_chunk(ci, parc):
        chunk_wait(parc)

        @pl.when(ci + 1 < NCHUNK)
        def _():
            chunk_load(ci + 1, 1 - parc)

        idxb, filtb, cntb = idx_b[parc], filt_b[parc], cnt_b[parc]
        for j in range(NBUF - 1):
            gather(idxb, j, j, sems[j])

        def body(bb, _):
            for j in range(NBUF):
                b = NBUF * bb + j
                nj = (j + NBUF - 1) % NBUF

                @pl.when(b + NBUF - 1 < CHUNK_BLKS)
                def _():
                    gather(idxb, b + NBUF - 1, nj, sems[nj])

                gather_wait(idxb, b, j, sems[j])
                compute_block(b, j, filtb, cntb)
            return 0

        lax.fori_loop(0, CHUNK_BLKS // NBUF, body, 0)
        pltpu.sync_copy(out_v, out.at[pl.ds(chunk_base(ci), CHUNK_BLKS)])

    chunk_load(0, 0)
    shard = PTS // NS

    @pl.when(c != HBM_CORE)
    def _():
        pltpu.sync_copy(table.at[pl.ds(s * shard, shard)],
                        tab_sh.at[pl.ds(s * shard, shard)])

    pltpu.sync_copy(dk.at[:], dkf_v)
    pltpu.sync_copy(rcp.at[:], rcp_v)
    plsc.subcore_barrier()

    def cpair(cc, _):
        do_chunk(2 * cc, 0)
        do_chunk(2 * cc + 1, 1)
        return 0

    lax.fori_loop(0, NCHUNK // 2, cpair, 0)


def _make_sc_call():
    mesh = plsc.VectorSubcoreMesh(core_axis_name="c", subcore_axis_name="s",
                                  num_cores=NC, num_subcores=NS)
    return pl.kernel(
        _sc_dw_kernel,
        out_type=jax.ShapeDtypeStruct((RBLOCKS, BLK * C), jnp.float32),
        mesh=mesh,
        compiler_params=pltpu.CompilerParams(use_tc_tiling_on_sc=False,
                                             needs_layout_passes=False),
        scratch_types=[
            pltpu.VMEM((BINS * C,), jnp.float32),
            pltpu.VMEM((CHUNK_BLKS, ROWS), jnp.int32),
            pltpu.VMEM((CHUNK_BLKS, ROWS), jnp.int32),
            pltpu.VMEM((CHUNK_BLKS, ROWS), jnp.int32),
            pltpu.VMEM((CHUNK_BLKS, ROWS), jnp.int32),
            pltpu.VMEM((CHUNK_PTS + 16,), jnp.int32),
            pltpu.VMEM((CHUNK_PTS + 16,), jnp.int32),
            pltpu.VMEM((K,), jnp.float32),
            pltpu.VMEM((NBUF, ROWS, C), jnp.float32),
            pltpu.VMEM((CHUNK_BLKS, BLK * C), jnp.float32),
            pltpu.VMEM_SHARED((PTS, C), jnp.float32),
            pltpu.SemaphoreType.DMA,
            pltpu.SemaphoreType.DMA,
        ] + [pltpu.SemaphoreType.DMA] * NBUF,
    )


PACK = BLK
PROWS = PTS // PACK


def _fold4(x):
    s = x[:, 0:O] + x[:, O:2 * O] + x[:, 2 * O:3 * O] + x[:, 3 * O:4 * O]
    return s, jnp.concatenate([s, s, s, s], axis=1)


def _tc_body(dw_ref, w_ref, b_ref, g_ref, be_ref, y_ref):
    x = dw_ref[:, :]
    y = jnp.dot(x, w_ref[:, :], preferred_element_type=jnp.float32) + b_ref[:, :]
    _, m = _fold4(jnp.sum(y, axis=0, keepdims=True) * (1.0 / PTS))
    d = y - m
    _, v = _fold4(jnp.sum(d * d, axis=0, keepdims=True) * (1.0 / PTS))
    scale = g_ref[:, :] / jnp.sqrt(v + 1e-5)
    y_ref[:, :] = jnp.maximum(d * scale + be_ref[:, :], 0.0)


def kernel(inputs, nn_index, nn_count, filt_index, depthwise_kernel, fc_w, fc_b, gamma, beta):
    table = inputs.reshape(PTS, C)
    offs = (jnp.arange(B, dtype=jnp.int32) * N)[:, None, None]
    idx2 = (nn_index.reshape(B, N * K // ROWS, ROWS) + offs).reshape(RBLOCKS, ROWS)
    filt2 = filt_index.reshape(RBLOCKS, ROWS)
    cnt1 = nn_count.reshape(PTS)
    dk2 = depthwise_kernel.reshape(BINS * C * M)
    rcp = 1.0 / jnp.arange(1, K + 1, dtype=jnp.float32)

    dw = _make_sc_call()(table, idx2, filt2, cnt1, dk2, rcp)

    w_bd = jnp.kron(jnp.eye(PACK, dtype=jnp.float32), fc_w)
    b_t = jnp.tile(fc_b, PACK).reshape(1, PACK * O)
    g_t = jnp.tile(gamma, PACK).reshape(1, PACK * O)
    be_t = jnp.tile(beta, PACK).reshape(1, PACK * O)
    y = pl.pallas_call(
        _tc_body,
        out_shape=jax.ShapeDtypeStruct((PROWS, PACK * O), jnp.float32),
    )(dw, w_bd, b_t, g_t, be_t)
    return y.reshape(B, N, O)
